# baseline (speedup 1.0000x reference)
import functools

import jax
import jax.numpy as jnp
import numpy as np
from jax.experimental import pallas as pl
from jax.experimental.pallas import tpu as pltpu
from jax.sharding import Mesh, PartitionSpec as P

NUM_IN = 768
CODE_C = 64
NUM_CODES = 1024
L = 24 * 24


def _vq_body(x_ref, v_in_ref, g_in_ref, b_in_ref, v_out_ref, g_out_ref,
             b_out_ref, cb_ref, cbT_ref, y_ref, idx_ref, loss_ref,
             wi_s, cbn_s, cbT_s, wo_s):
    @pl.when(pl.program_id(0) == 0)
    def _prep():
        v_in = v_in_ref[...]
        norm_in = jnp.sqrt(jnp.sum(v_in * v_in, axis=1, keepdims=True))
        wi_s[...] = (g_in_ref[...] * v_in / norm_in).astype(jnp.bfloat16)
        cb = cb_ref[...]
        cbn = cb / jnp.maximum(
            jnp.sqrt(jnp.sum(cb * cb, axis=1, keepdims=True)), 1e-8)
        cbn_s[...] = cbn.astype(jnp.bfloat16)
        cbT_s[...] = cbT_ref[...].astype(jnp.bfloat16)
        v_out = v_out_ref[...]
        norm_out = jnp.sqrt(jnp.sum(v_out * v_out, axis=1, keepdims=True))
        wo_s[...] = (g_out_ref[...] * v_out / norm_out).astype(jnp.bfloat16)

    x = x_ref[0]
    z_e = jnp.dot(wi_s[...], x,
                  preferred_element_type=jnp.float32) + b_in_ref[...]

    zn = jnp.maximum(
        jnp.sqrt(jnp.sum(z_e * z_e, axis=0, keepdims=True)), 1e-8)
    zhat = z_e / zn

    simT = jnp.dot(cbn_s[...], zhat.astype(jnp.bfloat16),
                   preferred_element_type=jnp.float32)

    idx = jnp.argmax(simT, axis=0).astype(jnp.int32)
    idx_ref[0, 0, :] = idx

    oh = (jax.lax.broadcasted_iota(jnp.int32, (NUM_CODES, L), 0)
          == idx[None, :]).astype(jnp.bfloat16)
    z_q = jnp.dot(cbT_s[...], oh,
                  preferred_element_type=jnp.float32)

    diff = z_q - z_e
    loss = jnp.sum(diff * diff) * (1.0 / (CODE_C * L))
    loss_ref[0, 0, :] = jnp.full((128,), loss, dtype=jnp.float32)

    y_ref[0] = jnp.dot(wo_s[...], z_q.astype(jnp.bfloat16),
                       preferred_element_type=jnp.float32) + b_out_ref[...]


@functools.partial(jax.jit, static_argnames=("interpret",))
def _vq_call(x3, v_in, g_in, b_in, v_out, g_out, b_out, codebook,
             interpret=False):
    n = x3.shape[0]
    cbT = codebook.T
    y, idx, loss = pl.pallas_call(
        _vq_body,
        grid=(n,),
        in_specs=[
            pl.BlockSpec((1, NUM_IN, L), lambda i: (i, 0, 0)),
            pl.BlockSpec((CODE_C, NUM_IN), lambda i: (0, 0)),
            pl.BlockSpec((CODE_C, 1), lambda i: (0, 0)),
            pl.BlockSpec((CODE_C, 1), lambda i: (0, 0)),
            pl.BlockSpec((NUM_IN, CODE_C), lambda i: (0, 0)),
            pl.BlockSpec((NUM_IN, 1), lambda i: (0, 0)),
            pl.BlockSpec((NUM_IN, 1), lambda i: (0, 0)),
            pl.BlockSpec((NUM_CODES, CODE_C), lambda i: (0, 0)),
            pl.BlockSpec((CODE_C, NUM_CODES), lambda i: (0, 0)),
        ],
        out_specs=[
            pl.BlockSpec((1, NUM_IN, L), lambda i: (i, 0, 0)),
            pl.BlockSpec((1, 1, L), lambda i: (i, 0, 0)),
            pl.BlockSpec((1, 1, 128), lambda i: (i, 0, 0)),
        ],
        out_shape=[
            jax.ShapeDtypeStruct((n, NUM_IN, L), jnp.float32),
            jax.ShapeDtypeStruct((n, 1, L), jnp.int32),
            jax.ShapeDtypeStruct((n, 1, 128), jnp.float32),
        ],
        scratch_shapes=[
            pltpu.VMEM((CODE_C, NUM_IN), jnp.bfloat16),
            pltpu.VMEM((NUM_CODES, CODE_C), jnp.bfloat16),
            pltpu.VMEM((CODE_C, NUM_CODES), jnp.bfloat16),
            pltpu.VMEM((NUM_IN, CODE_C), jnp.bfloat16),
        ],
        compiler_params=pltpu.CompilerParams(
            dimension_semantics=("parallel",)),
        interpret=interpret,
    )(x3, v_in, g_in[:, None], b_in[:, None], v_out, g_out[:, None],
      b_out[:, None], codebook, cbT)
    return y, idx, loss


def kernel(x, v_in, g_in, b_in, v_out, g_out, b_out, codebook):
    n = x.shape[0]
    dims = x.shape[2:]
    x3 = x.reshape(n, x.shape[1], -1).astype(jnp.bfloat16)
    devs = jax.devices()
    if len(devs) >= 2 and n % 2 == 0:
        mesh = Mesh(np.array(devs[:2]), ("d",))
        fn = jax.shard_map(
            _vq_call,
            mesh=mesh,
            in_specs=(P("d"), P(), P(), P(), P(), P(), P(), P()),
            out_specs=(P("d"), P("d"), P("d")),
            check_vma=False,
        )
        y3, idx, loss = fn(x3, v_in, g_in, b_in, v_out, g_out, b_out,
                           codebook)
    else:
        y3, idx, loss = _vq_call(x3, v_in, g_in, b_in, v_out, g_out, b_out,
                                 codebook)
    y = y3.reshape(x.shape)
    code_index = idx.reshape((n,) + tuple(dims))
    loss = loss[:, 0, 0]
    return (y, code_index, loss, loss)


# speedup vs baseline: 6.5765x; 6.5765x over previous
import functools

import jax
import jax.numpy as jnp
from jax.experimental import pallas as pl
from jax.experimental.pallas import tpu as pltpu
from jax.experimental.pallas import tpu_sc as plsc

NUM_IN = 768
CODE_C = 64
NUM_CODES = 1024
L = 24 * 24
GATHER_WIN = 128


def _stage_a_body(x_ref, v_in_ref, g_in_ref, b_in_ref, cb_ref,
                  z_e_ref, idx_ref, wi_s, cbn_s):
    @pl.when(pl.program_id(0) == 0)
    def _prep():
        v_in = v_in_ref[...]
        norm_in = jnp.sqrt(jnp.sum(v_in * v_in, axis=1, keepdims=True))
        wi_s[...] = (g_in_ref[...] * v_in / norm_in).astype(jnp.bfloat16)
        cb = cb_ref[...]
        cbn = cb / jnp.maximum(
            jnp.sqrt(jnp.sum(cb * cb, axis=1, keepdims=True)), 1e-8)
        cbn_s[...] = cbn.astype(jnp.bfloat16)

    x = x_ref[0]
    z_e = jnp.dot(wi_s[...], x,
                  preferred_element_type=jnp.float32) + b_in_ref[...]
    z_e_ref[0] = z_e

    zn = jnp.maximum(
        jnp.sqrt(jnp.sum(z_e * z_e, axis=0, keepdims=True)), 1e-8)
    zhat = z_e / zn

    simT = jnp.dot(cbn_s[...], zhat.astype(jnp.bfloat16),
                   preferred_element_type=jnp.float32)
    idx_ref[0, 0, :] = jnp.argmax(simT, axis=0).astype(jnp.int32)


def _stage_c_body(z_e_ref, zq_ref, v_out_ref, g_out_ref, b_out_ref,
                  y_ref, loss_ref, wo_s):
    @pl.when(pl.program_id(0) == 0)
    def _prep():
        v_out = v_out_ref[...]
        norm_out = jnp.sqrt(jnp.sum(v_out * v_out, axis=1, keepdims=True))
        wo_s[...] = (g_out_ref[...] * v_out / norm_out).astype(jnp.bfloat16)

    z_e = z_e_ref[0]
    z_q = jnp.transpose(zq_ref[0][:, :CODE_C])

    diff = z_q - z_e
    loss = jnp.sum(diff * diff) * (1.0 / (CODE_C * L))
    loss_ref[0, 0, :] = jnp.full((128,), loss, dtype=jnp.float32)

    y_ref[0] = jnp.dot(wo_s[...], z_q.astype(jnp.bfloat16),
                       preferred_element_type=jnp.float32) + b_out_ref[...]


def _sc_gather(codebook_pad, idx_flat):
    num_indices = idx_flat.shape[1]
    row = codebook_pad.shape[1]

    @pl.kernel(
        out_type=jax.ShapeDtypeStruct((num_indices, row),
                                      codebook_pad.dtype),
        mesh=plsc.VectorSubcoreMesh(core_axis_name="c",
                                    subcore_axis_name="s"),
    )
    def gather_kernel(cb_hbm, i_hbm, o_hbm):
        def body(i_vmem, o_vmem):
            pltpu.sync_copy(cb_hbm.at[i_vmem.at[0]], o_vmem)

        pltpu.emit_pipeline(
            body,
            grid=(num_indices // GATHER_WIN,),
            in_specs=[pl.BlockSpec((1, GATHER_WIN),
                                   index_map=lambda i: (0, i))],
            out_specs=[pl.BlockSpec((GATHER_WIN, row),
                                    index_map=lambda i: (i, 0))],
            core_axis_name="s",
            dimension_semantics=(pltpu.PARALLEL,),
        )(i_hbm, o_hbm)

    return gather_kernel(codebook_pad, idx_flat)


@jax.jit
def _vq_call(x3, v_in, g_in, b_in, v_out, g_out, b_out, codebook):
    n = x3.shape[0]
    z_e, idx = pl.pallas_call(
        _stage_a_body,
        grid=(n,),
        in_specs=[
            pl.BlockSpec((1, NUM_IN, L), lambda i: (i, 0, 0)),
            pl.BlockSpec((CODE_C, NUM_IN), lambda i: (0, 0)),
            pl.BlockSpec((CODE_C, 1), lambda i: (0, 0)),
            pl.BlockSpec((CODE_C, 1), lambda i: (0, 0)),
            pl.BlockSpec((NUM_CODES, CODE_C), lambda i: (0, 0)),
        ],
        out_specs=[
            pl.BlockSpec((1, CODE_C, L), lambda i: (i, 0, 0)),
            pl.BlockSpec((1, 1, L), lambda i: (i, 0, 0)),
        ],
        out_shape=[
            jax.ShapeDtypeStruct((n, CODE_C, L), jnp.float32),
            jax.ShapeDtypeStruct((n, 1, L), jnp.int32),
        ],
        scratch_shapes=[
            pltpu.VMEM((CODE_C, NUM_IN), jnp.bfloat16),
            pltpu.VMEM((NUM_CODES, CODE_C), jnp.bfloat16),
        ],
        compiler_params=pltpu.CompilerParams(
            dimension_semantics=("parallel",)),
    )(x3, v_in, g_in[:, None], b_in[:, None], codebook)

    cb_pad = jnp.pad(codebook, ((0, 0), (0, 128 - CODE_C)))
    zq = _sc_gather(cb_pad, idx.reshape(1, n * L))

    y, loss = pl.pallas_call(
        _stage_c_body,
        grid=(n,),
        in_specs=[
            pl.BlockSpec((1, CODE_C, L), lambda i: (i, 0, 0)),
            pl.BlockSpec((1, L, 128), lambda i: (i, 0, 0)),
            pl.BlockSpec((NUM_IN, CODE_C), lambda i: (0, 0)),
            pl.BlockSpec((NUM_IN, 1), lambda i: (0, 0)),
            pl.BlockSpec((NUM_IN, 1), lambda i: (0, 0)),
        ],
        out_specs=[
            pl.BlockSpec((1, NUM_IN, L), lambda i: (i, 0, 0)),
            pl.BlockSpec((1, 1, 128), lambda i: (i, 0, 0)),
        ],
        out_shape=[
            jax.ShapeDtypeStruct((n, NUM_IN, L), jnp.float32),
            jax.ShapeDtypeStruct((n, 1, 128), jnp.float32),
        ],
        scratch_shapes=[
            pltpu.VMEM((NUM_IN, CODE_C), jnp.bfloat16),
        ],
        compiler_params=pltpu.CompilerParams(
            dimension_semantics=("parallel",)),
    )(z_e, zq.reshape(n, L, 128), v_out, g_out[:, None], b_out[:, None])
    return y, idx, loss


def kernel(x, v_in, g_in, b_in, v_out, g_out, b_out, codebook):
    n = x.shape[0]
    dims = x.shape[2:]
    x3 = x.reshape(n, x.shape[1], -1).astype(jnp.bfloat16)
    y3, idx, loss = _vq_call(x3, v_in, g_in, b_in, v_out, g_out, b_out,
                             codebook)
    y = y3.reshape(x.shape)
    code_index = idx.reshape((n,) + tuple(dims))
    loss = loss[:, 0, 0]
    return (y, code_index, loss, loss)


# speedup vs baseline: 6.8846x; 1.0468x over previous
import functools

import jax
import jax.numpy as jnp
from jax.experimental import pallas as pl
from jax.experimental.pallas import tpu as pltpu
from jax.experimental.pallas import tpu_sc as plsc

NUM_IN = 768
CODE_C = 64
NUM_CODES = 1024
L = 24 * 24
GATHER_WIN = 128


def _stage_a_body(x_ref, v_in_ref, g_in_ref, b_in_ref, cb_ref,
                  z_e_ref, idx_ref, wi_s, cbn_s):
    @pl.when(pl.program_id(0) == 0)
    def _prep():
        v_in = v_in_ref[...]
        norm_in = jnp.sqrt(jnp.sum(v_in * v_in, axis=1, keepdims=True))
        wi_s[...] = (g_in_ref[...] * v_in / norm_in).astype(jnp.bfloat16)
        cb = cb_ref[...]
        cbn = cb / jnp.maximum(
            jnp.sqrt(jnp.sum(cb * cb, axis=1, keepdims=True)), 1e-8)
        cbn_s[...] = cbn.astype(jnp.bfloat16)

    x = x_ref[0]
    z_e = jnp.dot(wi_s[...], x,
                  preferred_element_type=jnp.float32) + b_in_ref[...]
    z_e_ref[0] = z_e

    zn = jnp.maximum(
        jnp.sqrt(jnp.sum(z_e * z_e, axis=0, keepdims=True)), 1e-8)
    zhat = z_e / zn

    simT = jnp.dot(cbn_s[...], zhat.astype(jnp.bfloat16),
                   preferred_element_type=jnp.float32)
    idx_ref[0, 0, :] = jnp.argmax(simT, axis=0).astype(jnp.int32)


def _stage_c_body(z_e_ref, zq_ref, v_out_ref, g_out_ref, b_out_ref,
                  y_ref, loss_ref, wo_s):
    @pl.when(pl.program_id(0) == 0)
    def _prep():
        v_out = v_out_ref[...]
        norm_out = jnp.sqrt(jnp.sum(v_out * v_out, axis=1, keepdims=True))
        wo_s[...] = (g_out_ref[...] * v_out / norm_out).astype(jnp.bfloat16)

    z_e = z_e_ref[0]
    z_q = jnp.transpose(zq_ref[0][:, :CODE_C])

    diff = z_q - z_e
    loss = jnp.sum(diff * diff) * (1.0 / (CODE_C * L))
    loss_ref[0, 0, :] = jnp.full((128,), loss, dtype=jnp.float32)

    y_ref[0] = jnp.dot(wo_s[...], z_q.astype(jnp.bfloat16),
                       preferred_element_type=jnp.float32) + b_out_ref[...]


def _sc_gather(codebook_pad, idx_flat):
    num_indices = idx_flat.shape[1]
    row = codebook_pad.shape[1]

    @pl.kernel(
        out_type=jax.ShapeDtypeStruct((num_indices, row),
                                      codebook_pad.dtype),
        mesh=plsc.VectorSubcoreMesh(core_axis_name="c",
                                    subcore_axis_name="s"),
    )
    def gather_kernel(cb_hbm, i_hbm, o_hbm):
        def body(i_vmem, o_vmem):
            pltpu.sync_copy(cb_hbm.at[i_vmem.at[0]], o_vmem)

        pltpu.emit_pipeline(
            body,
            grid=(num_indices // GATHER_WIN,),
            in_specs=[pl.BlockSpec((1, GATHER_WIN),
                                   index_map=lambda i: (0, i))],
            out_specs=[pl.BlockSpec((GATHER_WIN, row),
                                    index_map=lambda i: (i, 0))],
            core_axis_name=("c", "s"),
            dimension_semantics=(pltpu.PARALLEL,),
        )(i_hbm, o_hbm)

    return gather_kernel(codebook_pad, idx_flat)


@jax.jit
def _vq_call(x3, v_in, g_in, b_in, v_out, g_out, b_out, codebook):
    n = x3.shape[0]
    z_e, idx = pl.pallas_call(
        _stage_a_body,
        grid=(n,),
        in_specs=[
            pl.BlockSpec((1, NUM_IN, L), lambda i: (i, 0, 0)),
            pl.BlockSpec((CODE_C, NUM_IN), lambda i: (0, 0)),
            pl.BlockSpec((CODE_C, 1), lambda i: (0, 0)),
            pl.BlockSpec((CODE_C, 1), lambda i: (0, 0)),
            pl.BlockSpec((NUM_CODES, CODE_C), lambda i: (0, 0)),
        ],
        out_specs=[
            pl.BlockSpec((1, CODE_C, L), lambda i: (i, 0, 0)),
            pl.BlockSpec((1, 1, L), lambda i: (i, 0, 0)),
        ],
        out_shape=[
            jax.ShapeDtypeStruct((n, CODE_C, L), jnp.float32),
            jax.ShapeDtypeStruct((n, 1, L), jnp.int32),
        ],
        scratch_shapes=[
            pltpu.VMEM((CODE_C, NUM_IN), jnp.bfloat16),
            pltpu.VMEM((NUM_CODES, CODE_C), jnp.bfloat16),
        ],
        compiler_params=pltpu.CompilerParams(
            dimension_semantics=("parallel",)),
    )(x3, v_in, g_in[:, None], b_in[:, None], codebook)

    cb_pad = jnp.pad(codebook, ((0, 0), (0, 128 - CODE_C)))
    zq = _sc_gather(cb_pad, idx.reshape(1, n * L))

    y, loss = pl.pallas_call(
        _stage_c_body,
        grid=(n,),
        in_specs=[
            pl.BlockSpec((1, CODE_C, L), lambda i: (i, 0, 0)),
            pl.BlockSpec((1, L, 128), lambda i: (i, 0, 0)),
            pl.BlockSpec((NUM_IN, CODE_C), lambda i: (0, 0)),
            pl.BlockSpec((NUM_IN, 1), lambda i: (0, 0)),
            pl.BlockSpec((NUM_IN, 1), lambda i: (0, 0)),
        ],
        out_specs=[
            pl.BlockSpec((1, NUM_IN, L), lambda i: (i, 0, 0)),
            pl.BlockSpec((1, 1, 128), lambda i: (i, 0, 0)),
        ],
        out_shape=[
            jax.ShapeDtypeStruct((n, NUM_IN, L), jnp.float32),
            jax.ShapeDtypeStruct((n, 1, 128), jnp.float32),
        ],
        scratch_shapes=[
            pltpu.VMEM((NUM_IN, CODE_C), jnp.bfloat16),
        ],
        compiler_params=pltpu.CompilerParams(
            dimension_semantics=("parallel",)),
    )(z_e, zq.reshape(n, L, 128), v_out, g_out[:, None], b_out[:, None])
    return y, idx, loss


def kernel(x, v_in, g_in, b_in, v_out, g_out, b_out, codebook):
    n = x.shape[0]
    dims = x.shape[2:]
    x3 = x.reshape(n, x.shape[1], -1).astype(jnp.bfloat16)
    y3, idx, loss = _vq_call(x3, v_in, g_in, b_in, v_out, g_out, b_out,
                             codebook)
    y = y3.reshape(x.shape)
    code_index = idx.reshape((n,) + tuple(dims))
    loss = loss[:, 0, 0]
    return (y, code_index, loss, loss)


# speedup vs baseline: 6.8926x; 1.0012x over previous
import functools

import jax
import jax.numpy as jnp
from jax.experimental import pallas as pl
from jax.experimental.pallas import tpu as pltpu
from jax.experimental.pallas import tpu_sc as plsc

NUM_IN = 768
CODE_C = 64
NUM_CODES = 1024
L = 24 * 24
GATHER_WIN = 128


def _stage_a_body(x_ref, v_in_ref, g_in_ref, b_in_ref, cb_ref,
                  z_e_ref, idx_ref, wi_s, cbn_s):
    @pl.when(pl.program_id(0) == 0)
    def _prep():
        v_in = v_in_ref[...]
        norm_in = jnp.sqrt(jnp.sum(v_in * v_in, axis=1, keepdims=True))
        wi_s[...] = (g_in_ref[...] * v_in / norm_in).astype(jnp.bfloat16)
        cb = cb_ref[...]
        cbn = cb / jnp.maximum(
            jnp.sqrt(jnp.sum(cb * cb, axis=1, keepdims=True)), 1e-8)
        cbn_s[...] = cbn.astype(jnp.bfloat16)

    x = x_ref[0]
    z_e = jnp.dot(wi_s[...], x,
                  preferred_element_type=jnp.float32) + b_in_ref[...]
    z_e_ref[0] = z_e.astype(jnp.bfloat16)

    zn = jnp.maximum(
        jnp.sqrt(jnp.sum(z_e * z_e, axis=0, keepdims=True)), 1e-8)
    zhat = z_e / zn

    simT = jnp.dot(cbn_s[...], zhat.astype(jnp.bfloat16),
                   preferred_element_type=jnp.float32)
    idx_ref[0, 0, :] = jnp.argmax(simT, axis=0).astype(jnp.int32)


def _stage_c_body(z_e_ref, zq_ref, v_out_ref, g_out_ref, b_out_ref,
                  y_ref, loss_ref, wo_s):
    @pl.when(pl.program_id(0) == 0)
    def _prep():
        v_out = v_out_ref[...]
        norm_out = jnp.sqrt(jnp.sum(v_out * v_out, axis=1, keepdims=True))
        wo_s[...] = (g_out_ref[...] * v_out / norm_out).astype(jnp.bfloat16)

    z_e = z_e_ref[0].astype(jnp.float32)
    z_q = jnp.transpose(zq_ref[0][:, :CODE_C])
    z_qb = z_q.astype(jnp.bfloat16)

    diff = z_q - z_e
    loss = jnp.sum(diff * diff) * (1.0 / (CODE_C * L))
    loss_ref[0, 0, :] = jnp.full((128,), loss, dtype=jnp.float32)

    y_ref[0] = jnp.dot(wo_s[...], z_qb,
                       preferred_element_type=jnp.float32) + b_out_ref[...]


def _sc_gather(codebook_pad, idx_flat):
    num_indices = idx_flat.shape[1]
    row = codebook_pad.shape[1]

    @pl.kernel(
        out_type=jax.ShapeDtypeStruct((num_indices, row),
                                      codebook_pad.dtype),
        mesh=plsc.VectorSubcoreMesh(core_axis_name="c",
                                    subcore_axis_name="s"),
    )
    def gather_kernel(cb_hbm, i_hbm, o_hbm):
        def body(i_vmem, o_vmem):
            pltpu.sync_copy(cb_hbm.at[i_vmem.at[0]], o_vmem)

        pltpu.emit_pipeline(
            body,
            grid=(num_indices // GATHER_WIN,),
            in_specs=[pl.BlockSpec((1, GATHER_WIN),
                                   index_map=lambda i: (0, i))],
            out_specs=[pl.BlockSpec((GATHER_WIN, row),
                                    index_map=lambda i: (i, 0))],
            core_axis_name=("c", "s"),
            dimension_semantics=(pltpu.PARALLEL,),
        )(i_hbm, o_hbm)

    return gather_kernel(codebook_pad, idx_flat)


@jax.jit
def _vq_call(x3, v_in, g_in, b_in, v_out, g_out, b_out, codebook):
    n = x3.shape[0]
    z_e, idx = pl.pallas_call(
        _stage_a_body,
        grid=(n,),
        in_specs=[
            pl.BlockSpec((1, NUM_IN, L), lambda i: (i, 0, 0)),
            pl.BlockSpec((CODE_C, NUM_IN), lambda i: (0, 0)),
            pl.BlockSpec((CODE_C, 1), lambda i: (0, 0)),
            pl.BlockSpec((CODE_C, 1), lambda i: (0, 0)),
            pl.BlockSpec((NUM_CODES, CODE_C), lambda i: (0, 0)),
        ],
        out_specs=[
            pl.BlockSpec((1, CODE_C, L), lambda i: (i, 0, 0)),
            pl.BlockSpec((1, 1, L), lambda i: (i, 0, 0)),
        ],
        out_shape=[
            jax.ShapeDtypeStruct((n, CODE_C, L), jnp.bfloat16),
            jax.ShapeDtypeStruct((n, 1, L), jnp.int32),
        ],
        scratch_shapes=[
            pltpu.VMEM((CODE_C, NUM_IN), jnp.bfloat16),
            pltpu.VMEM((NUM_CODES, CODE_C), jnp.bfloat16),
        ],
        compiler_params=pltpu.CompilerParams(
            dimension_semantics=("parallel",)),
    )(x3, v_in, g_in[:, None], b_in[:, None], codebook)

    cb_pad = jnp.pad(codebook, ((0, 0), (0, 128 - CODE_C)))
    zq = _sc_gather(cb_pad, idx.reshape(1, n * L))

    y, loss = pl.pallas_call(
        _stage_c_body,
        grid=(n,),
        in_specs=[
            pl.BlockSpec((1, CODE_C, L), lambda i: (i, 0, 0)),
            pl.BlockSpec((1, L, 128), lambda i: (i, 0, 0)),
            pl.BlockSpec((NUM_IN, CODE_C), lambda i: (0, 0)),
            pl.BlockSpec((NUM_IN, 1), lambda i: (0, 0)),
            pl.BlockSpec((NUM_IN, 1), lambda i: (0, 0)),
        ],
        out_specs=[
            pl.BlockSpec((1, NUM_IN, L), lambda i: (i, 0, 0)),
            pl.BlockSpec((1, 1, 128), lambda i: (i, 0, 0)),
        ],
        out_shape=[
            jax.ShapeDtypeStruct((n, NUM_IN, L), jnp.float32),
            jax.ShapeDtypeStruct((n, 1, 128), jnp.float32),
        ],
        scratch_shapes=[
            pltpu.VMEM((NUM_IN, CODE_C), jnp.bfloat16),
        ],
        compiler_params=pltpu.CompilerParams(
            dimension_semantics=("parallel",)),
    )(z_e, zq.reshape(n, L, 128), v_out, g_out[:, None], b_out[:, None])
    return y, idx, loss


def kernel(x, v_in, g_in, b_in, v_out, g_out, b_out, codebook):
    n = x.shape[0]
    dims = x.shape[2:]
    x3 = x.reshape(n, x.shape[1], -1).astype(jnp.bfloat16)
    y3, idx, loss = _vq_call(x3, v_in, g_in, b_in, v_out, g_out, b_out,
                             codebook)
    y = y3.reshape(x.shape)
    code_index = idx.reshape((n,) + tuple(dims))
    loss = loss[:, 0, 0]
    return (y, code_index, loss, loss)
